# baseline (speedup 1.0000x reference)
import functools

import jax
import jax.numpy as jnp
from jax import lax
from jax.experimental import pallas as pl
from jax.experimental.pallas import tpu as pltpu
from jax.experimental.pallas import tpu_sc as plsc

_NS = 16
_D = 128
_S = 32
_NPAD = 10240
_CN = 4
_RING = 4
_SPLIT = (512, 128)


def _make_gather_mean(npw):
    nchunk = npw // _CN
    nhq = nchunk // 8
    mesh = plsc.VectorSubcoreMesh(core_axis_name="c", subcore_axis_name="s")

    @functools.partial(
        pl.kernel,
        out_type=jax.ShapeDtypeStruct((_NS * npw, _D), jnp.float32),
        mesh=mesh,
        scratch_types=[
            pltpu.VMEM((npw * _S,), jnp.int32),
            [pltpu.VMEM((_CN * _S, _D), jnp.float32)
             for _ in range(_RING)],
            [pltpu.VMEM((4 * _CN, _D), jnp.float32)
             for _ in range(2)],
            [pltpu.SemaphoreType.DMA for _ in range(_RING)],
            [pltpu.SemaphoreType.DMA for _ in range(2)],
        ],
    )
    def k(table_hbm, idx_hbm, out_hbm, idx_v, rows, outb, gsem, osem):
        cid = lax.axis_index("c")
        sid = lax.axis_index("s")
        base = sid * npw

        def start(c, rows_v, sem):
            pltpu.async_copy(
                table_hbm.at[idx_v.at[pl.ds(c * (_CN * _S), _CN * _S)]],
                rows_v, sem)

        def wait_gather(k_slot):
            pltpu.make_async_copy(
                table_hbm.at[idx_v.at[pl.ds(0, _CN * _S)]],
                rows[k_slot], gsem[k_slot]).wait()

        def reduce_chunk(rows_v, outb_v, orow):
            for j in range(_CN):
                def row_body(r, accs):
                    row = j * _S + r * 4
                    for u in range(4):
                        accs = tuple(
                            accs[g] + rows_v[row + u, pl.ds(g * 16, 16)]
                            for g in range(8))
                    return accs
                accs = lax.fori_loop(
                    0, _S // 4, row_body,
                    tuple(jnp.zeros((16,), jnp.float32) for _ in range(8)))
                for g in range(8):
                    outb_v[orow + j, pl.ds(g * 16, 16)] = accs[g] * (1.0 / _S)

        @pl.when(cid == 0)
        def _():
            pltpu.sync_copy(idx_hbm.at[pl.ds(base * _S, npw * _S)], idx_v)
            for k_slot in range(_RING):
                start(jnp.int32(k_slot), rows[k_slot], gsem[k_slot])

            def hq_body(hq, carry):
                for qi in range(2):
                    q = hq * 2 + qi

                    @pl.when(hq > 0)
                    def _():
                        pltpu.make_async_copy(
                            outb[qi], out_hbm.at[pl.ds(0, 4 * _CN)],
                            osem[qi]).wait()

                    for k_slot in range(_RING):
                        c = q * 4 + k_slot
                        wait_gather(k_slot)
                        reduce_chunk(rows[k_slot], outb[qi], k_slot * _CN)
                        start(jnp.minimum(c + _RING, nchunk - 1),
                              rows[k_slot], gsem[k_slot])
                    pltpu.async_copy(
                        outb[qi],
                        out_hbm.at[pl.ds(base + q * (4 * _CN), 4 * _CN)],
                        osem[qi])
                return carry

            lax.fori_loop(0, nhq, hq_body, 0)
            for k_slot in range(_RING):
                wait_gather(k_slot)
            for qi in range(2):
                pltpu.make_async_copy(
                    outb[qi], out_hbm.at[pl.ds(0, 4 * _CN)], osem[qi]).wait()

    return k


_GATHER_A = _make_gather_mean(_SPLIT[0])
_GATHER_B = _make_gather_mean(_SPLIT[1])
_CUT = _NS * _SPLIT[0]


def _gather_mean(table, idx_flat):
    ga = _GATHER_A(table, idx_flat[:_CUT * _S])
    gb = _GATHER_B(table, idx_flat[_CUT * _S:])
    return jnp.concatenate([ga, gb], axis=0)


def _sage_linear(a, b, wa, wb, bias, relu):
    npad = a.shape[0]
    bm = 512

    def mm(a_ref, b_ref, wa_ref, wb_ref, bias_ref, o_ref):
        acc = jnp.dot(a_ref[...], wa_ref[...],
                      preferred_element_type=jnp.float32)
        acc = acc + jnp.dot(b_ref[...], wb_ref[...],
                            preferred_element_type=jnp.float32)
        acc = acc + bias_ref[...]
        if relu:
            acc = jnp.maximum(acc, 0.0)
        o_ref[...] = acc

    return pl.pallas_call(
        mm,
        grid=(npad // bm,),
        in_specs=[
            pl.BlockSpec((bm, _D), lambda i: (i, 0)),
            pl.BlockSpec((bm, _D), lambda i: (i, 0)),
            pl.BlockSpec((_D, _D), lambda i: (0, 0)),
            pl.BlockSpec((_D, _D), lambda i: (0, 0)),
            pl.BlockSpec((1, _D), lambda i: (0, 0)),
        ],
        out_specs=pl.BlockSpec((bm, _D), lambda i: (i, 0)),
        out_shape=jax.ShapeDtypeStruct((npad, _D), jnp.float32),
    )(a, b, wa, wb, bias)


def kernel(x, adj, sampled_neighbors, W1, b1, W2, b2):
    n, d = x.shape
    xp = jnp.zeros((_NPAD, d), x.dtype).at[:n].set(x)
    nbrp = jnp.concatenate(
        [sampled_neighbors,
         jnp.zeros((2, _NPAD - n, _S), sampled_neighbors.dtype)], axis=1)
    idx0 = nbrp[0].reshape(-1)
    idx1 = nbrp[1].reshape(-1)
    w1a, w1b = W1[:d], W1[d:]
    w2a, w2b = W2[:d], W2[d:]

    g1 = _gather_mean(xp, idx0)
    h1 = _sage_linear(xp, g1, w1a, w1b, b1.reshape(1, d), relu=True)
    g2 = _gather_mean(h1, idx1)
    h2 = _sage_linear(h1, g2, w2a, w2b, b2.reshape(1, d), relu=False)
    return h2[:n]


# speedup vs baseline: 1.4347x; 1.4347x over previous
import functools

import jax
import jax.numpy as jnp
from jax import lax
from jax.experimental import pallas as pl
from jax.experimental.pallas import tpu as pltpu
from jax.experimental.pallas import tpu_sc as plsc

_NC, _NS = 2, 16
_NW = _NC * _NS
_D = 128
_S = 32
_NPAD = 10240
_NPW = _NPAD // _NW
_CN = 4
_RING = 4
_NCHUNK = _NPW // _CN
_NHQ = _NCHUNK // 8


def _gather_mean(table, idx_flat):
    mesh = plsc.VectorSubcoreMesh(core_axis_name="c", subcore_axis_name="s")

    @functools.partial(
        pl.kernel,
        out_type=jax.ShapeDtypeStruct((_NPAD, _D), jnp.float32),
        mesh=mesh,
        scratch_types=[
            pltpu.VMEM((_NPW * _S,), jnp.int32),
            [pltpu.VMEM((_CN * _S, _D), jnp.float32)
             for _ in range(_RING)],
            [pltpu.VMEM((4 * _CN, _D), jnp.float32)
             for _ in range(2)],
            [pltpu.SemaphoreType.DMA for _ in range(_RING)],
            [pltpu.SemaphoreType.DMA for _ in range(2)],
        ],
    )
    def k(table_hbm, idx_hbm, out_hbm, idx_v, rows, outb, gsem, osem):
        wid = lax.axis_index("s") * _NC + lax.axis_index("c")
        base = wid * _NPW
        pltpu.sync_copy(idx_hbm.at[pl.ds(base * _S, _NPW * _S)], idx_v)

        def start(c, rows_v, sem):
            pltpu.async_copy(
                table_hbm.at[idx_v.at[pl.ds(c * (_CN * _S), _CN * _S)]],
                rows_v, sem)

        def wait_gather(k_slot):
            pltpu.make_async_copy(
                table_hbm.at[idx_v.at[pl.ds(0, _CN * _S)]],
                rows[k_slot], gsem[k_slot]).wait()

        def reduce_chunk(rows_v, outb_v, orow):
            for j in range(_CN):
                def row_body(r, accs):
                    row = j * _S + r * 4
                    for u in range(4):
                        accs = tuple(
                            accs[g] + rows_v[row + u, pl.ds(g * 16, 16)]
                            for g in range(8))
                    return accs
                accs = lax.fori_loop(
                    0, _S // 4, row_body,
                    tuple(jnp.zeros((16,), jnp.float32) for _ in range(8)))
                for g in range(8):
                    outb_v[orow + j, pl.ds(g * 16, 16)] = accs[g] * (1.0 / _S)

        for k_slot in range(_RING):
            start(jnp.int32(k_slot), rows[k_slot], gsem[k_slot])

        def hq_body(hq, carry):
            for qi in range(2):
                q = hq * 2 + qi

                @pl.when(hq > 0)
                def _():
                    pltpu.make_async_copy(
                        outb[qi], out_hbm.at[pl.ds(0, 4 * _CN)],
                        osem[qi]).wait()

                for k_slot in range(_RING):
                    c = q * 4 + k_slot
                    wait_gather(k_slot)
                    reduce_chunk(rows[k_slot], outb[qi], k_slot * _CN)
                    start(jnp.minimum(c + _RING, _NCHUNK - 1),
                          rows[k_slot], gsem[k_slot])
                pltpu.async_copy(
                    outb[qi],
                    out_hbm.at[pl.ds(base + q * (4 * _CN), 4 * _CN)],
                    osem[qi])
            return carry

        lax.fori_loop(0, _NHQ, hq_body, 0)
        for k_slot in range(_RING):
            wait_gather(k_slot)
        for qi in range(2):
            pltpu.make_async_copy(
                outb[qi], out_hbm.at[pl.ds(0, 4 * _CN)], osem[qi]).wait()

    return k(table, idx_flat)


def _sage_linear(a, b, wa, wb, bias, relu):
    npad = a.shape[0]
    bm = 512

    def mm(a_ref, b_ref, wa_ref, wb_ref, bias_ref, o_ref):
        acc = jnp.dot(a_ref[...], wa_ref[...],
                      preferred_element_type=jnp.float32)
        acc = acc + jnp.dot(b_ref[...], wb_ref[...],
                            preferred_element_type=jnp.float32)
        acc = acc + bias_ref[...]
        if relu:
            acc = jnp.maximum(acc, 0.0)
        o_ref[...] = acc

    return pl.pallas_call(
        mm,
        grid=(npad // bm,),
        in_specs=[
            pl.BlockSpec((bm, _D), lambda i: (i, 0)),
            pl.BlockSpec((bm, _D), lambda i: (i, 0)),
            pl.BlockSpec((_D, _D), lambda i: (0, 0)),
            pl.BlockSpec((_D, _D), lambda i: (0, 0)),
            pl.BlockSpec((1, _D), lambda i: (0, 0)),
        ],
        out_specs=pl.BlockSpec((bm, _D), lambda i: (i, 0)),
        out_shape=jax.ShapeDtypeStruct((npad, _D), jnp.float32),
    )(a, b, wa, wb, bias)


def kernel(x, adj, sampled_neighbors, W1, b1, W2, b2):
    n, d = x.shape
    xp = jnp.zeros((_NPAD, d), x.dtype).at[:n].set(x)
    nbrp = jnp.concatenate(
        [sampled_neighbors,
         jnp.zeros((2, _NPAD - n, _S), sampled_neighbors.dtype)], axis=1)
    idx0 = nbrp[0].reshape(-1)
    idx1 = nbrp[1].reshape(-1)
    w1a, w1b = W1[:d], W1[d:]
    w2a, w2b = W2[:d], W2[d:]

    g1 = _gather_mean(xp, idx0)
    h1 = _sage_linear(xp, g1, w1a, w1b, b1.reshape(1, d), relu=True)
    g2 = _gather_mean(h1, idx1)
    h2 = _sage_linear(h1, g2, w2a, w2b, b2.reshape(1, d), relu=False)
    return h2[:n]


# speedup vs baseline: 5.6251x; 3.9209x over previous
import functools

import jax
import jax.numpy as jnp
from jax import lax
from jax.experimental import pallas as pl
from jax.experimental.pallas import tpu as pltpu
from jax.experimental.pallas import tpu_sc as plsc

_NC, _NS = 2, 16
_NW = _NC * _NS
_D = 128
_S = 32
_NPAD = 10240
_NPW = _NPAD // _NW
_CN = 4
_RING = 2
_NCHUNK = _NPW // _CN
_NT2 = _NCHUNK // 4


def _gather_mean(table, idx_flat):
    mesh = plsc.VectorSubcoreMesh(core_axis_name="c", subcore_axis_name="s")

    @functools.partial(
        pl.kernel,
        out_type=jax.ShapeDtypeStruct((_NPAD, _D), jnp.float32),
        mesh=mesh,
        scratch_types=[
            pltpu.VMEM((_NPW * _S,), jnp.int32),
            pltpu.VMEM_SHARED((_NPAD, _D), jnp.float32),
            [pltpu.VMEM((_CN * _S, _D), jnp.float32)
             for _ in range(_RING)],
            [pltpu.VMEM((2 * _CN, _D), jnp.float32)
             for _ in range(2)],
            [pltpu.SemaphoreType.DMA for _ in range(_RING)],
            [pltpu.SemaphoreType.DMA for _ in range(2)],
        ],
    )
    def k(table_hbm, idx_hbm, out_hbm, idx_v, tab_s, rows, outb, gsem, osem):
        wid = lax.axis_index("s") * _NC + lax.axis_index("c")
        sid = lax.axis_index("s")
        base = wid * _NPW

        stg = _NPAD // _NS
        pltpu.sync_copy(table_hbm.at[pl.ds(sid * stg, stg)],
                        tab_s.at[pl.ds(sid * stg, stg)])
        pltpu.sync_copy(idx_hbm.at[pl.ds(base * _S, _NPW * _S)], idx_v)
        plsc.subcore_barrier()

        def start(c, rows_v, sem):
            pltpu.async_copy(
                tab_s.at[idx_v.at[pl.ds(c * (_CN * _S), _CN * _S)]],
                rows_v, sem)

        def wait_gather(k_slot):
            pltpu.make_async_copy(
                tab_s.at[idx_v.at[pl.ds(0, _CN * _S)]],
                rows[k_slot], gsem[k_slot]).wait()

        def reduce_chunk(rows_v, outb_v, orow):
            for j in range(_CN):
                def row_body(r, accs):
                    row = j * _S + r * 4
                    for u in range(4):
                        accs = tuple(
                            accs[g] + rows_v[row + u, pl.ds(g * 16, 16)]
                            for g in range(8))
                    return accs
                accs = lax.fori_loop(
                    0, _S // 4, row_body,
                    tuple(jnp.zeros((16,), jnp.float32) for _ in range(8)))
                for g in range(8):
                    outb_v[orow + j, pl.ds(g * 16, 16)] = accs[g] * (1.0 / _S)

        for k_slot in range(_RING):
            start(jnp.int32(k_slot), rows[k_slot], gsem[k_slot])

        def t2_body(t2, carry):
            for s in range(2):
                p = t2 * 2 + s

                @pl.when(t2 > 0)
                def _():
                    pltpu.make_async_copy(
                        outb[s], out_hbm.at[pl.ds(0, 2 * _CN)],
                        osem[s]).wait()

                for k_slot in range(_RING):
                    c = p * 2 + k_slot
                    wait_gather(k_slot)
                    reduce_chunk(rows[k_slot], outb[s], k_slot * _CN)
                    start(jnp.minimum(c + _RING, _NCHUNK - 1),
                          rows[k_slot], gsem[k_slot])
                pltpu.async_copy(
                    outb[s],
                    out_hbm.at[pl.ds(base + p * (2 * _CN), 2 * _CN)],
                    osem[s])
            return carry

        lax.fori_loop(0, _NT2, t2_body, 0)
        for k_slot in range(_RING):
            wait_gather(k_slot)
        for s in range(2):
            pltpu.make_async_copy(
                outb[s], out_hbm.at[pl.ds(0, 2 * _CN)], osem[s]).wait()

    return k(table, idx_flat)


def _sage_linear(a, b, wa, wb, bias, relu):
    npad = a.shape[0]
    bm = 512

    def mm(a_ref, b_ref, wa_ref, wb_ref, bias_ref, o_ref):
        acc = jnp.dot(a_ref[...], wa_ref[...],
                      preferred_element_type=jnp.float32)
        acc = acc + jnp.dot(b_ref[...], wb_ref[...],
                            preferred_element_type=jnp.float32)
        acc = acc + bias_ref[...]
        if relu:
            acc = jnp.maximum(acc, 0.0)
        o_ref[...] = acc

    return pl.pallas_call(
        mm,
        grid=(npad // bm,),
        in_specs=[
            pl.BlockSpec((bm, _D), lambda i: (i, 0)),
            pl.BlockSpec((bm, _D), lambda i: (i, 0)),
            pl.BlockSpec((_D, _D), lambda i: (0, 0)),
            pl.BlockSpec((_D, _D), lambda i: (0, 0)),
            pl.BlockSpec((1, _D), lambda i: (0, 0)),
        ],
        out_specs=pl.BlockSpec((bm, _D), lambda i: (i, 0)),
        out_shape=jax.ShapeDtypeStruct((npad, _D), jnp.float32),
    )(a, b, wa, wb, bias)


def kernel(x, adj, sampled_neighbors, W1, b1, W2, b2):
    n, d = x.shape
    xp = jnp.zeros((_NPAD, d), x.dtype).at[:n].set(x)
    nbrp = jnp.concatenate(
        [sampled_neighbors,
         jnp.zeros((2, _NPAD - n, _S), sampled_neighbors.dtype)], axis=1)
    idx0 = nbrp[0].reshape(-1)
    idx1 = nbrp[1].reshape(-1)
    w1a, w1b = W1[:d], W1[d:]
    w2a, w2b = W2[:d], W2[d:]

    g1 = _gather_mean(xp, idx0)
    h1 = _sage_linear(xp, g1, w1a, w1b, b1.reshape(1, d), relu=True)
    g2 = _gather_mean(h1, idx1)
    h2 = _sage_linear(h1, g2, w2a, w2b, b2.reshape(1, d), relu=False)
    return h2[:n]
